# baseline (speedup 1.0000x reference)
import functools

import jax
import jax.numpy as jnp
from jax import lax
from jax.experimental import pallas as pl
from jax.experimental.pallas import tpu as pltpu
from jax.experimental.pallas import tpu_sc as plsc

F = 26
V = 100000
D = 16
B = 16384
IN = F * D
INP = 512
NUM_CROSS = 3
NUM_LINEAR = 2

NC = 2
NS = 16
NW = NC * NS
R = B * F
BW = B // NW
G = 16
CH = BW // G


def _sc_gather(table, idx2d):
    mesh = plsc.VectorSubcoreMesh(core_axis_name="c", subcore_axis_name="s")

    @functools.partial(
        pl.kernel,
        out_type=jax.ShapeDtypeStruct((B, INP), jnp.float32),
        mesh=mesh,
        scratch_types=[
            pltpu.VMEM((BW, F), jnp.int32),
            pltpu.VMEM((G, F, D), jnp.float32),
            pltpu.VMEM((G, INP), jnp.float32),
            pltpu.SemaphoreType.DMA,
        ],
        compiler_params=pltpu.CompilerParams(use_tc_tiling_on_sc=False),
    )
    def gather_kernel(table_hbm, idx_hbm, out_hbm, idx_v, rows_v, xbuf, sem):
        wid = lax.axis_index("s") * NC + lax.axis_index("c")
        pltpu.sync_copy(idx_hbm.at[pl.ds(wid * BW, BW)], idx_v)

        zeros = jnp.zeros((16,), jnp.float32)

        def zrow(g, carry):
            for t in range(IN // 16, INP // 16):
                xbuf[g, pl.ds(t * 16, 16)] = zeros
            return carry

        lax.fori_loop(0, G, zrow, 0)

        def chunk(c, carry):
            copies = [
                pltpu.make_async_copy(
                    table_hbm.at[idx_v.at[c * G + g]],
                    rows_v.at[g],
                    sem,
                )
                for g in range(G)
            ]
            for cp in copies:
                cp.start()
            for cp in copies:
                cp.wait()

            def repack(g, carry2):
                for t in range(F):
                    xbuf[g, pl.ds(t * D, D)] = rows_v[g, t]
                return carry2

            lax.fori_loop(0, G, repack, 0)
            orow = wid * BW + c * G
            pltpu.sync_copy(xbuf, out_hbm.at[pl.ds(orow, G)])
            return carry

        lax.fori_loop(0, CH, chunk, 0)

    return gather_kernel(table, idx2d)


def _bdot(a, b_ref):
    return jnp.dot(a.astype(jnp.bfloat16), b_ref,
                   preferred_element_type=jnp.float32)


def _dcn_body(x_ref, dummy_ref, wcin_ref, bcin_ref, wcout_ref, bcout_ref,
              wlin_ref, blin_ref, wout_ref, bout_ref, out_ref):
    x0 = x_ref[...]
    x = x0
    for i in range(NUM_CROSS):
        h = _bdot(x, wcin_ref[i]) + bcin_ref[i]
        h = jax.nn.gelu(_bdot(h, wcout_ref[i]) + bcout_ref[i])
        x = x0 * h + x
    for i in range(NUM_LINEAR):
        x = jax.nn.gelu(_bdot(x, wlin_ref[i]) + blin_ref[i])
    out_ref[...] = _bdot(x, wout_ref[...]) + bout_ref[...]


def _pad_w(w):
    pad = [(0, 0)] * (w.ndim - 2) + [(0, INP - IN), (0, INP - IN)]
    return jnp.pad(w.astype(jnp.bfloat16), pad)


def _pad_b(b):
    pad = [(0, 0)] * (b.ndim - 1) + [(0, INP - IN)]
    return jnp.pad(b, pad).reshape(b.shape[:-1] + (1, INP))


def _tc_dense(X, DUMMY_TBL, W_cin, b_cin, W_cout, b_cout, W_lin, b_lin,
              W_out, b_out):
    BT = 512
    grid = (B // BT,)
    full = lambda shape: pl.BlockSpec(shape, lambda i: (0,) * len(shape))
    return pl.pallas_call(
        _dcn_body,
        grid=grid,
        in_specs=[
            pl.BlockSpec((BT, INP), lambda i: (i, 0)),
            pl.BlockSpec((8, 128), lambda i: (0, 0)),
            full((NUM_CROSS, INP, INP)),
            full((NUM_CROSS, 1, INP)),
            full((NUM_CROSS, INP, INP)),
            full((NUM_CROSS, 1, INP)),
            full((NUM_LINEAR, INP, INP)),
            full((NUM_LINEAR, 1, INP)),
            full((INP, 1)),
            full((1, 1)),
        ],
        out_specs=pl.BlockSpec((BT, 1), lambda i: (i, 0)),
        out_shape=jax.ShapeDtypeStruct((B, 1), jnp.float32),
        compiler_params=pltpu.CompilerParams(
            dimension_semantics=("arbitrary",),
        ),
    )(X, DUMMY_TBL, _pad_w(W_cin), _pad_b(b_cin), _pad_w(W_cout), _pad_b(b_cout),
      _pad_w(W_lin), _pad_b(b_lin),
      jnp.pad(W_out.astype(jnp.bfloat16), ((0, INP - IN), (0, 0))),
      b_out.reshape(1, 1))


def kernel(inputs, emb_tables, W_cin, b_cin, W_cout, b_cout, W_lin, b_lin,
           W_out, b_out):
    offs = (jnp.arange(F, dtype=jnp.int32) * V)[None, :]
    idx2d = inputs + offs
    X = _sc_gather(emb_tables.reshape(F * V, D), idx2d)
    return _tc_dense(X, emb_tables.reshape(F * V * D // 128, 128),
                     W_cin, b_cin, W_cout, b_cout, W_lin, b_lin,
                     W_out, b_out)


# speedup vs baseline: 1.6510x; 1.6510x over previous
import functools

import jax
import jax.numpy as jnp
from jax import lax
from jax.experimental import pallas as pl
from jax.experimental.pallas import tpu as pltpu
from jax.experimental.pallas import tpu_sc as plsc

F = 26
V = 100000
D = 16
B = 16384
IN = F * D
NUM_CROSS = 3
NUM_LINEAR = 2

NC = 2
NS = 16
NW = NC * NS
R = B * F
RW = R // NW
S = 128
SPW = RW // S
K = 13
CH = SPW // K


def _sc_gather(table_flat, idx2d):
    mesh = plsc.VectorSubcoreMesh(core_axis_name="c", subcore_axis_name="s")

    @functools.partial(
        pl.kernel,
        out_type=jax.ShapeDtypeStruct((R, D), jnp.float32),
        mesh=mesh,
        scratch_types=[
            pltpu.VMEM((SPW, S), jnp.int32),
            pltpu.VMEM((2, K * S, D), jnp.float32),
            pltpu.SemaphoreType.DMA,
        ],
        compiler_params=pltpu.CompilerParams(use_tc_tiling_on_sc=False),
    )
    def gather_kernel(table_hbm, idx_hbm, out_hbm, idx_v, rows_v, sem):
        wid = lax.axis_index("s") * NC + lax.axis_index("c")
        pltpu.sync_copy(idx_hbm.at[pl.ds(wid * SPW, SPW)], idx_v)

        def fire(c, buf):
            for j in range(K):
                pltpu.make_async_copy(
                    table_hbm.at[idx_v.at[c * K + j]],
                    rows_v.at[buf, pl.ds(j * S, S)],
                    sem,
                ).start()

        def drain_and_flush(c, buf):
            pltpu.make_async_copy(
                out_hbm.at[pl.ds(0, K * S)], rows_v.at[buf], sem
            ).wait()
            orow = wid * RW + c * (K * S)
            pltpu.sync_copy(rows_v.at[buf], out_hbm.at[pl.ds(orow, K * S)])

        fire(0, 0)

        def chunk(c, carry):
            fire(c, c % 2)
            drain_and_flush(c - 1, (c - 1) % 2)
            return carry

        lax.fori_loop(1, CH, chunk, 0)
        drain_and_flush(CH - 1, (CH - 1) % 2)

    return gather_kernel(table_flat, idx2d)


def _bdot(a, b_ref):
    return jnp.dot(a.astype(jnp.bfloat16), b_ref,
                   preferred_element_type=jnp.float32)


def _dcn_body(x_ref, wcin_ref, bcin_ref, wcout_ref, bcout_ref,
              wlin_ref, blin_ref, wout_ref, bout_ref, out_ref):
    x0 = x_ref[...]
    x = x0
    for i in range(NUM_CROSS):
        h = _bdot(x, wcin_ref[i]) + bcin_ref[i]
        h = jax.nn.gelu(_bdot(h, wcout_ref[i]) + bcout_ref[i])
        x = x0 * h + x
    for i in range(NUM_LINEAR):
        x = jax.nn.gelu(_bdot(x, wlin_ref[i]) + blin_ref[i])
    out_ref[...] = _bdot(x, wout_ref[...]) + bout_ref[...]


def _tc_dense(X, W_cin, b_cin, W_cout, b_cout, W_lin, b_lin, W_out, b_out):
    BT = 1024
    grid = (B // BT,)
    full = lambda shape: pl.BlockSpec(shape, lambda i: (0,) * len(shape))
    return pl.pallas_call(
        _dcn_body,
        grid=grid,
        in_specs=[
            pl.BlockSpec((BT, IN), lambda i: (i, 0)),
            full((NUM_CROSS, IN, IN)),
            full((NUM_CROSS, 1, IN)),
            full((NUM_CROSS, IN, IN)),
            full((NUM_CROSS, 1, IN)),
            full((NUM_LINEAR, IN, IN)),
            full((NUM_LINEAR, 1, IN)),
            full((IN, 1)),
            full((1, 1)),
        ],
        out_specs=pl.BlockSpec((BT, 1), lambda i: (i, 0)),
        out_shape=jax.ShapeDtypeStruct((B, 1), jnp.float32),
        compiler_params=pltpu.CompilerParams(
            dimension_semantics=("arbitrary",),
        ),
    )(X, W_cin.astype(jnp.bfloat16), b_cin.reshape(NUM_CROSS, 1, IN),
      W_cout.astype(jnp.bfloat16), b_cout.reshape(NUM_CROSS, 1, IN),
      W_lin.astype(jnp.bfloat16), b_lin.reshape(NUM_LINEAR, 1, IN),
      W_out.astype(jnp.bfloat16), b_out.reshape(1, 1))


def kernel(inputs, emb_tables, W_cin, b_cin, W_cout, b_cout, W_lin, b_lin,
           W_out, b_out):
    offs = (jnp.arange(F, dtype=jnp.int32) * V)[None, :]
    idx2d = (inputs + offs).reshape(R // S, S)
    X = _sc_gather(emb_tables.reshape(F * V, D), idx2d).reshape(B, IN)
    return _tc_dense(X, W_cin, b_cin, W_cout, b_cout, W_lin, b_lin,
                     W_out, b_out)
